# baseline (speedup 1.0000x reference)
import functools

import jax
import jax.numpy as jnp
from jax import lax
from jax.experimental import pallas as pl
from jax.experimental.pallas import tpu as pltpu
from jax.experimental.pallas import tpu_sc as plsc

NC = 2
NS = 16
NW = NC * NS
CB = 8
HP = 56
EP = 128


@functools.lru_cache(maxsize=None)
def _build(batch: int, hist: int, vocab: int, embed: int):
    assert batch % (NW * CB) == 0
    rows_per_w = batch // NW
    n_chunks = rows_per_w // CB
    mesh = plsc.VectorSubcoreMesh(core_axis_name="c", subcore_axis_name="s")

    @functools.partial(
        pl.kernel,
        mesh=mesh,
        compiler_params=pltpu.CompilerParams(use_tc_tiling_on_sc=False),
        out_type=jax.ShapeDtypeStruct((NW, n_chunks, CB, HP, EP), jnp.float32),
        scratch_types=[
            pltpu.VMEM((rows_per_w, EP), jnp.int32),
            pltpu.VMEM((2, CB, HP, embed), jnp.float32),
            pltpu.SemaphoreType.DMA,
            pltpu.SemaphoreType.DMA,
        ],
    )
    def emb_kernel(idx_hbm, table_hbm, out_hbm, idx_v, stage_v, sem_g, sem_o):
        wid = lax.axis_index("s") * NC + lax.axis_index("c")
        pltpu.sync_copy(idx_hbm.at[pl.ds(wid * rows_per_w, rows_per_w)], idx_v)

        def fire_gathers(c):
            s = lax.rem(c, 2)
            for bb in range(CB):
                pltpu.async_copy(
                    table_hbm.at[idx_v.at[c * CB + bb, pl.ds(0, HP)]],
                    stage_v.at[s, bb],
                    sem_g,
                )

        def fire_write(c):
            s = lax.rem(c, 2)
            pltpu.async_copy(
                stage_v.at[s],
                out_hbm.at[wid, c, slice(None), slice(None), pl.ds(0, embed)],
                sem_o,
            )

        def drain_g(n):
            for _ in range(n):
                pltpu.make_async_copy(
                    out_hbm.at[wid, 0, 0, slice(None), pl.ds(0, embed)],
                    stage_v.at[0, 0],
                    sem_g,
                ).wait()

        def drain_o(n):
            for _ in range(n):
                pltpu.make_async_copy(
                    stage_v.at[0],
                    out_hbm.at[wid, 0, slice(None), slice(None), pl.ds(0, embed)],
                    sem_o,
                ).wait()

        fire_gathers(0)
        drain_g(CB)
        fire_write(0)
        fire_gathers(1)

        @pl.loop(1, n_chunks - 1)
        def _(c):
            drain_g(CB)
            drain_o(1)
            fire_write(c)
            fire_gathers(c + 1)

        drain_g(CB)
        drain_o(1)
        fire_write(n_chunks - 1)
        drain_o(1)

    return emb_kernel


def kernel(x, vectors):
    batch, hist = x.shape
    vocab, embed = vectors.shape
    idx = jnp.pad(x.astype(jnp.int32), ((0, 0), (0, EP - hist)))
    out = _build(batch, hist, vocab, embed)(idx, vectors)
    return out.reshape(batch, HP, EP)[:, :hist, :embed]


# speedup vs baseline: 4.2694x; 4.2694x over previous
import functools

import jax
import jax.numpy as jnp
from jax import lax
from jax.experimental import pallas as pl
from jax.experimental.pallas import tpu as pltpu
from jax.experimental.pallas import tpu_sc as plsc

NC = 2
NS = 16
NW = NC * NS
CB = 8
HP = 56
EP = 128


@functools.lru_cache(maxsize=None)
def _build(batch: int, hist: int, vocab: int, embed: int):
    assert batch % (NW * CB) == 0
    rows_per_w = batch // NW
    n_chunks = rows_per_w // CB
    mesh = plsc.VectorSubcoreMesh(core_axis_name="c", subcore_axis_name="s")

    @functools.partial(
        pl.kernel,
        mesh=mesh,
        compiler_params=pltpu.CompilerParams(use_tc_tiling_on_sc=False),
        out_type=jax.ShapeDtypeStruct((NW, n_chunks, CB, HP, EP), jnp.float32),
        scratch_types=[
            pltpu.VMEM((rows_per_w, hist), jnp.int32),
            pltpu.VMEM((2, CB, hist, embed), jnp.float32),
            pltpu.SemaphoreType.DMA,
            pltpu.SemaphoreType.DMA,
        ],
    )
    def emb_kernel(idx_hbm, table_hbm, out_hbm, idx_v, stage_v, sem_g, sem_o):
        wid = lax.axis_index("s") * NC + lax.axis_index("c")
        pltpu.sync_copy(idx_hbm.at[wid], idx_v)

        def fire_gathers(c):
            s = lax.rem(c, 2)
            for bb in range(CB):
                pltpu.async_copy(
                    table_hbm.at[idx_v.at[c * CB + bb]],
                    stage_v.at[s, bb],
                    sem_g,
                )

        def fire_write(c):
            s = lax.rem(c, 2)
            pltpu.async_copy(
                stage_v.at[s],
                out_hbm.at[wid, c, slice(None), pl.ds(0, hist), pl.ds(0, embed)],
                sem_o,
            )

        def drain_g(n):
            for _ in range(n):
                pltpu.make_async_copy(
                    out_hbm.at[wid, 0, 0, pl.ds(0, hist), pl.ds(0, embed)],
                    stage_v.at[0, 0],
                    sem_g,
                ).wait()

        def drain_o(n):
            for _ in range(n):
                pltpu.make_async_copy(
                    stage_v.at[0],
                    out_hbm.at[wid, 0, slice(None), pl.ds(0, hist), pl.ds(0, embed)],
                    sem_o,
                ).wait()

        fire_gathers(0)
        drain_g(CB)
        fire_write(0)
        fire_gathers(1)

        @pl.loop(1, n_chunks - 1)
        def _(c):
            drain_g(CB)
            drain_o(1)
            fire_write(c)
            fire_gathers(c + 1)

        drain_g(CB)
        drain_o(1)
        fire_write(n_chunks - 1)
        drain_o(1)

    return emb_kernel


def kernel(x, vectors):
    batch, hist = x.shape
    vocab, embed = vectors.shape
    idx = x.astype(jnp.int32).reshape(NW, batch // NW, hist)
    out = _build(batch, hist, vocab, embed)(idx, vectors + 0.0)
    return out.reshape(batch, HP, EP)[:, :hist, :embed]


# speedup vs baseline: 4.3930x; 1.0289x over previous
import functools

import jax
import jax.numpy as jnp
from jax import lax
from jax.experimental import pallas as pl
from jax.experimental.pallas import tpu as pltpu
from jax.experimental.pallas import tpu_sc as plsc

NC = 2
NS = 16
NW = NC * NS
CB = 4
NSET = 4
HP = 56
EP = 128


@functools.lru_cache(maxsize=None)
def _build(batch: int, hist: int, vocab: int, embed: int):
    assert batch % (NW * CB) == 0
    rows_per_w = batch // NW
    n_chunks = rows_per_w // CB
    assert n_chunks > NSET
    mesh = plsc.VectorSubcoreMesh(core_axis_name="c", subcore_axis_name="s")

    @functools.partial(
        pl.kernel,
        mesh=mesh,
        compiler_params=pltpu.CompilerParams(use_tc_tiling_on_sc=False),
        out_type=jax.ShapeDtypeStruct((NW, n_chunks, CB, HP, EP), jnp.float32),
        scratch_types=[
            pltpu.VMEM((rows_per_w, hist), jnp.int32),
            pltpu.VMEM((NSET, CB, hist, embed), jnp.float32),
            pltpu.SemaphoreType.DMA,
            pltpu.SemaphoreType.DMA,
        ],
    )
    def emb_kernel(idx_hbm, table_hbm, out_hbm, idx_v, stage_v, sem_g, sem_o):
        wid = lax.axis_index("s") * NC + lax.axis_index("c")
        pltpu.sync_copy(idx_hbm.at[wid], idx_v)

        def fire_gathers(c):
            s = lax.rem(c, NSET)
            for bb in range(CB):
                pltpu.async_copy(
                    table_hbm.at[idx_v.at[c * CB + bb]],
                    stage_v.at[s, bb],
                    sem_g,
                )

        def fire_write(c):
            s = lax.rem(c, NSET)
            pltpu.async_copy(
                stage_v.at[s],
                out_hbm.at[wid, c, slice(None), pl.ds(0, hist), pl.ds(0, embed)],
                sem_o,
            )

        def drain_g(n):
            for _ in range(n):
                pltpu.make_async_copy(
                    out_hbm.at[wid, 0, 0, pl.ds(0, hist), pl.ds(0, embed)],
                    stage_v.at[0, 0],
                    sem_g,
                ).wait()

        def drain_o(n):
            for _ in range(n):
                pltpu.make_async_copy(
                    stage_v.at[0],
                    out_hbm.at[wid, 0, slice(None), pl.ds(0, hist), pl.ds(0, embed)],
                    sem_o,
                ).wait()

        for c in range(NSET - 1):
            fire_gathers(c)

        drain_g(CB)
        fire_write(0)
        fire_gathers(NSET - 1)

        @pl.loop(1, n_chunks - (NSET - 1))
        def _(c):
            drain_g(CB)
            drain_o(1)
            fire_write(c)
            fire_gathers(c + NSET - 1)

        @pl.loop(n_chunks - (NSET - 1), n_chunks)
        def _(c):
            drain_g(CB)
            fire_write(c)

        drain_o(NSET)

    return emb_kernel


def kernel(x, vectors):
    batch, hist = x.shape
    vocab, embed = vectors.shape
    idx = x.astype(jnp.int32).reshape(NW, batch // NW, hist)
    out = _build(batch, hist, vocab, embed)(idx, vectors)
    return out.reshape(batch, HP, EP)[:, :hist, :embed]
